# baseline (speedup 1.0000x reference)
import functools

import jax
import jax.numpy as jnp
from jax import lax
from jax.experimental import pallas as pl
from jax.experimental.pallas import tpu as pltpu
from jax.experimental.pallas import tpu_sc as plsc

NC = 2
NS = 16
NW = NC * NS
CHUNK = 128


def _build(B, D):
    b_per_w = B // NW
    nch = b_per_w // CHUNK
    mesh = plsc.VectorSubcoreMesh(
        core_axis_name="c", subcore_axis_name="s",
        num_cores=NC, num_subcores=NS)

    @functools.partial(
        pl.kernel,
        out_type=(jax.ShapeDtypeStruct((B, D), jnp.float32),
                  jax.ShapeDtypeStruct((B, D), jnp.float32)),
        mesh=mesh,
        scratch_types=[
            pltpu.VMEM((nch, CHUNK), jnp.int32),
            pltpu.VMEM((nch, CHUNK), jnp.int32),
            pltpu.VMEM((CHUNK, D), jnp.float32),
            pltpu.SemaphoreType.DMA,
        ],
        compiler_params=pltpu.CompilerParams(use_tc_tiling_on_sc=False),
    )
    def k(src_t, trg_t, sidx, tidx, out_s, out_t, sidx_v, tidx_v, rows_v, sem):
        wid = lax.axis_index("s") * NC + lax.axis_index("c")
        base = wid * b_per_w
        pltpu.sync_copy(sidx.at[wid], sidx_v)
        pltpu.sync_copy(tidx.at[wid], tidx_v)

        def body_s(c, carry):
            pltpu.async_copy(src_t.at[sidx_v.at[c]], rows_v, sem).wait()
            pltpu.sync_copy(rows_v, out_s.at[pl.ds(base + c * CHUNK, CHUNK)])
            return carry
        lax.fori_loop(0, nch, body_s, 0)

        def body_t(c, carry):
            pltpu.async_copy(trg_t.at[tidx_v.at[c]], rows_v, sem).wait()
            pltpu.sync_copy(rows_v, out_t.at[pl.ds(base + c * CHUNK, CHUNK)])
            return carry
        lax.fori_loop(0, nch, body_t, 0)

    return k


def kernel(src_table, trg_table, src_indices, trg_indices):
    batch, seq = src_indices.shape
    D = src_table.shape[1]
    B = batch * seq
    sidx = src_indices.reshape(NW, B // NW // CHUNK, CHUNK).astype(jnp.int32)
    tidx = trg_indices.reshape(NW, B // NW // CHUNK, CHUNK).astype(jnp.int32)
    out_s, out_t = _build(B, D)(src_table, trg_table, sidx, tidx)
    return (out_s.reshape(batch, seq, D), out_t.reshape(batch, seq, D))


# speedup vs baseline: 1.0504x; 1.0504x over previous
import functools

import jax
import jax.numpy as jnp
from jax import lax
from jax.experimental import pallas as pl
from jax.experimental.pallas import tpu as pltpu
from jax.experimental.pallas import tpu_sc as plsc

NC = 2
NS = 16
NW = NC * NS
CHUNK = 128
CPG = 2
GROUP = CHUNK * CPG
NBUF = 5


def _build(B, D):
    b_per_w = B // NW
    nch = b_per_w // CHUNK
    ngrp = nch // CPG
    niter = ngrp // NBUF
    mesh = plsc.VectorSubcoreMesh(
        core_axis_name="c", subcore_axis_name="s",
        num_cores=NC, num_subcores=NS)

    @functools.partial(
        pl.kernel,
        out_type=(jax.ShapeDtypeStruct((B, D), jnp.float32),
                  jax.ShapeDtypeStruct((B, D), jnp.float32)),
        mesh=mesh,
        scratch_types=[
            pltpu.VMEM((nch, CHUNK), jnp.int32),
            pltpu.VMEM((nch, CHUNK), jnp.int32),
            pltpu.VMEM((NBUF, GROUP, D), jnp.float32),
        ] + [pltpu.SemaphoreType.DMA] * (2 * NBUF),
        compiler_params=pltpu.CompilerParams(use_tc_tiling_on_sc=False),
    )
    def k(src_t, trg_t, sidx, tidx, out_s, out_t, sidx_v, tidx_v, buf, *sems):
        gsem = sems[:NBUF]
        ssem = sems[NBUF:]
        wid = lax.axis_index("s") * NC + lax.axis_index("c")
        base = wid * b_per_w
        pltpu.sync_copy(sidx.at[wid], sidx_v)
        pltpu.sync_copy(tidx.at[wid], tidx_v)

        def issue_group(table, idx_v, g, b):
            for j in range(CPG):
                pltpu.async_copy(
                    table.at[idx_v.at[g * CPG + j]],
                    buf.at[b, pl.ds(j * CHUNK, CHUNK)],
                    gsem[b])

        def wait_group(table, b):
            pltpu.make_async_copy(
                table.at[pl.ds(0, GROUP)], buf.at[b], gsem[b]).wait()

        def wait_scatter(out, b):
            pltpu.make_async_copy(
                buf.at[b], out.at[pl.ds(0, GROUP)], ssem[b]).wait()

        def run_table(table, idx_v, out, drain_prev):
            for b in range(NBUF):
                if drain_prev:
                    wait_scatter(out, b)
                issue_group(table, idx_v, b, b)

            def body(i, carry):
                for b in range(NBUF):
                    g = i * NBUF + b
                    wait_group(table, b)
                    pltpu.async_copy(
                        buf.at[b],
                        out.at[pl.ds(base + g * GROUP, GROUP)],
                        ssem[b])

                    @pl.when(i < niter - 1)
                    def _():
                        wait_scatter(out, b)
                        issue_group(table, idx_v, g + NBUF, b)
                return carry
            lax.fori_loop(0, niter, body, 0)

        run_table(src_t, sidx_v, out_s, False)
        run_table(trg_t, tidx_v, out_t, True)
        for b in range(NBUF):
            wait_scatter(out_t, b)

    return k


def kernel(src_table, trg_table, src_indices, trg_indices):
    batch, seq = src_indices.shape
    D = src_table.shape[1]
    B = batch * seq
    sidx = src_indices.reshape(NW, B // NW // CHUNK, CHUNK).astype(jnp.int32)
    tidx = trg_indices.reshape(NW, B // NW // CHUNK, CHUNK).astype(jnp.int32)
    out_s, out_t = _build(B, D)(src_table, trg_table, sidx, tidx)
    return (out_s.reshape(batch, seq, D), out_t.reshape(batch, seq, D))
